# baseline (speedup 1.0000x reference)
import dataclasses
import functools

import jax
import jax.numpy as jnp
import numpy as np
from jax import lax
from jax.experimental import pallas as pl
from jax.experimental.pallas import tpu as pltpu
from jax.experimental.pallas import tpu_sc as plsc

N = 10000
E = 320000
C = 128
H = 64
NCORE = 2
NSUB = 16
LANES = 16

EPS = E // NSUB
B = 32
NCHUNK = EPS // B
ZROW = 8
NZCH = N // ZROW
DROW = 40
NDCH = N // DROW

MBLK = 400

_PERM192 = np.concatenate(
    [g * 32 + np.array([(m // 2) if m % 2 == 0 else 16 + (m // 2)
                        for m in range(32)]) for g in range(6)])


def _ycat_body(xr_ref, xi_ref, w_ref, o_ref):
    w = w_ref[0]
    a = jnp.dot(xr_ref[...], w, preferred_element_type=jnp.float32)
    b = jnp.dot(xi_ref[...], w, preferred_element_type=jnp.float32)
    o_ref[0, :, 0:192] = a.astype(jnp.bfloat16)
    o_ref[0, :, 192:384] = b.astype(jnp.bfloat16)


def _project(x_real, x_imag, wsel):
    return pl.pallas_call(
        _ycat_body,
        grid=(NCORE, N // MBLK),
        in_specs=[
            pl.BlockSpec((MBLK, C), lambda c, n: (n, 0)),
            pl.BlockSpec((MBLK, C), lambda c, n: (n, 0)),
            pl.BlockSpec((1, C, 3 * H), lambda c, n: (c, 0, 0)),
        ],
        out_specs=pl.BlockSpec((1, MBLK, 8 * H), lambda c, n: (c, n, 0)),
        out_shape=jax.ShapeDtypeStruct((NCORE, N, 8 * H), jnp.bfloat16),
    )(x_real, x_imag, wsel)


_sc_mesh = plsc.VectorSubcoreMesh(core_axis_name="c", subcore_axis_name="s")

_sc_params = pltpu.CompilerParams()
if "needs_layout_passes" in pltpu.CompilerParams.__dataclass_fields__:
    _sc_params = dataclasses.replace(_sc_params, needs_layout_passes=False)


@functools.partial(
    pl.kernel,
    out_type=jax.ShapeDtypeStruct((NCORE * N, C), jnp.float32),
    mesh=_sc_mesh,
    scratch_types=[
        pltpu.VMEM((6, B), jnp.float32),
        pltpu.VMEM((6, B), jnp.float32),
        pltpu.VMEM((6, B), jnp.float32),
        pltpu.VMEM((6, B), jnp.float32),
        pltpu.VMEM((B,), jnp.int32),
        pltpu.VMEM((B,), jnp.int32),
        pltpu.VMEM((B,), jnp.int32),
        pltpu.VMEM((B,), jnp.int32),
        pltpu.VMEM((B,), jnp.int32),
        pltpu.VMEM((B,), jnp.int32),
        pltpu.VMEM((B, 4 * H), jnp.int32),
        pltpu.VMEM((B, 4 * H), jnp.int32),
        pltpu.VMEM((B, C), jnp.float32),
        pltpu.VMEM((B, C), jnp.float32),
        pltpu.VMEM((ZROW, C), jnp.float32),
        pltpu.VMEM_SHARED((N, C), jnp.float32),
        pltpu.SemaphoreType.DMA,
        pltpu.SemaphoreType.DMA,
        pltpu.SemaphoreType.DMA,
        pltpu.SemaphoreType.DMA,
        pltpu.SemaphoreType.DMA,
    ],
    compiler_params=_sc_params,
)
def _sc_spmm(ycat_hbm, lr_hbm, li_hbm, src_hbm, dst_hbm, out_hbm,
             recv0, recv1, recv2, recv3, srcv0, srcv1,
             dstv0, dstv1, dstv2, dstv3,
             rows0, rows1, outv0, outv1, zbuf, acc,
             gsem0, gsem1, ssem0, ssem1, msem):
    c = lax.axis_index("c")
    s = lax.axis_index("s")

    recvs = [recv0, recv1, recv2, recv3]
    srcvs = [srcv0, srcv1]
    dstvs = [dstv0, dstv1, dstv2, dstv3]
    rowss = [rows0, rows1]
    outvs = [outv0, outv1]
    gsems = [gsem0, gsem1]
    ssems = [ssem0, ssem1]

    zero16 = jnp.zeros((LANES,), jnp.float32)

    @pl.loop(0, ZROW)
    def _(r):
        for kk in range(C // LANES):
            zbuf[r, pl.ds(kk * LANES, LANES)] = zero16

    @pl.loop(s, NZCH, step=NSUB)
    def _(ch):
        pltpu.sync_copy(zbuf, acc.at[pl.ds(ch * ZROW, ZROW)])

    plsc.subcore_barrier()

    icol = [jnp.full((LANES,), i, dtype=jnp.int32) for i in range(6)]
    coffv = jnp.full((LANES,), c * N, dtype=jnp.int32)
    base0 = s * EPS

    def _base(t):
        return jnp.minimum(base0 + t * B, E - B)

    def meta_issue(t, recv_b, srcv_b, dstv_b):
        base = _base(t)
        pltpu.async_copy(src_hbm.at[pl.ds(base, B)], srcv_b, msem)
        pltpu.async_copy(dst_hbm.at[pl.ds(base, B)], dstv_b, msem)
        for i in range(3):
            pltpu.async_copy(lr_hbm.at[i, pl.ds(base, B)], recv_b.at[i], msem)
            pltpu.async_copy(li_hbm.at[i, pl.ds(base, B)], recv_b.at[i + 3], msem)

    def meta_wait(t, recv_b, srcv_b, dstv_b):
        base = _base(t)
        pltpu.make_async_copy(src_hbm.at[pl.ds(base, B)], srcv_b, msem).wait()
        pltpu.make_async_copy(dst_hbm.at[pl.ds(base, B)], dstv_b, msem).wait()
        for i in range(3):
            pltpu.make_async_copy(lr_hbm.at[i, pl.ds(base, B)], recv_b.at[i], msem).wait()
            pltpu.make_async_copy(li_hbm.at[i, pl.ds(base, B)], recv_b.at[i + 3], msem).wait()
        for hh in range(B // LANES):
            srcv_b[pl.ds(hh * LANES, LANES)] = (
                srcv_b[pl.ds(hh * LANES, LANES)] + coffv)

    def gather_issue(srcv_b, rows_b, sem):
        pltpu.async_copy(ycat_hbm.at[srcv_b], rows_b, sem)

    def gather_wait(srcv_b, rows_b, sem):
        pltpu.make_async_copy(ycat_hbm.at[srcv_b], rows_b, sem).wait()

    def scatter_issue(outv_b, dstv_b, sem):
        pltpu.async_copy(outv_b, acc.at[dstv_b], sem, add=True)

    def scatter_wait(outv_b, dstv_b, sem):
        pltpu.make_async_copy(outv_b, acc.at[dstv_b], sem).wait()

    def compute(recv_b, rows_b, outv_b):
        @plsc.parallel_loop(0, B, 1, unroll=2)
        def _(j):
            jvec = jnp.full((LANES,), j, dtype=jnp.int32)
            w0 = plsc.load_gather(recv_b, [icol[0], jvec])
            w1 = plsc.load_gather(recv_b, [icol[1], jvec])
            w2 = plsc.load_gather(recv_b, [icol[2], jvec])
            w3 = plsc.load_gather(recv_b, [icol[3], jvec])
            w4 = plsc.load_gather(recv_b, [icol[4], jvec])
            w5 = plsc.load_gather(recv_b, [icol[5], jvec])
            HW = H // 2
            for k in range(H // 32):
                o = k * LANES
                def _ld(off):
                    v = plsc.bitcast(rows_b[j, pl.ds(off, LANES)], jnp.bfloat16)
                    return plsc.unpack(v, format=plsc.PackFormat.INTERLEAVED)
                yr0a, yr0b = _ld(o)
                yr1a, yr1b = _ld(HW + o)
                yr2a, yr2b = _ld(2 * HW + o)
                yi0a, yi0b = _ld(3 * HW + o)
                yi1a, yi1b = _ld(4 * HW + o)
                yi2a, yi2b = _ld(5 * HW + o)
                rea = (w0 * yr0a + w1 * yr1a + w2 * yr2a
                       - w3 * yi0a - w4 * yi1a - w5 * yi2a)
                reb = (w0 * yr0b + w1 * yr1b + w2 * yr2b
                       - w3 * yi0b - w4 * yi1b - w5 * yi2b)
                ima = (w3 * yr0a + w4 * yr1a + w5 * yr2a
                       + w0 * yi0a + w1 * yi1a + w2 * yi2a)
                imb = (w3 * yr0b + w4 * yr1b + w5 * yr2b
                       + w0 * yi0b + w1 * yi1b + w2 * yi2b)
                outv_b[j, pl.ds(o, LANES)] = rea
                outv_b[j, pl.ds(o + LANES, LANES)] = reb
                outv_b[j, pl.ds(H + o, LANES)] = ima
                outv_b[j, pl.ds(H + o + LANES, LANES)] = imb

    for t in range(2):
        meta_issue(t, recvs[t], srcvs[t], dstvs[t])
        meta_wait(t, recvs[t], srcvs[t], dstvs[t])
        gather_issue(srcvs[t], rowss[t], gsems[t])

    @pl.loop(0, NCHUNK - 1, step=4)
    def _(t0):
        for b4 in range(4):
            t = t0 + b4
            b2 = b4 % 2
            gather_wait(srcvs[b2], rowss[b2], gsems[b2])

            @pl.when(t >= 2)
            def _():
                scatter_wait(outvs[b2], dstvs[(b4 + 2) % 4], ssems[b2])

            meta_issue(t + 2, recvs[(b4 + 2) % 4], srcvs[b2],
                       dstvs[(b4 + 2) % 4])
            compute(recvs[b4], rowss[b2], outvs[b2])
            scatter_issue(outvs[b2], dstvs[b4], ssems[b2])
            meta_wait(t + 2, recvs[(b4 + 2) % 4], srcvs[b2],
                      dstvs[(b4 + 2) % 4])
            gather_issue(srcvs[b2], rowss[b2], gsems[b2])

    gather_wait(srcvs[0], rowss[0], gsems[0])
    scatter_wait(outvs[0], dstvs[2], ssems[0])
    compute(recvs[0], rowss[0], outvs[0])
    scatter_issue(outvs[0], dstvs[0], ssems[0])
    gather_wait(srcvs[1], rowss[1], gsems[1])
    scatter_wait(outvs[1], dstvs[3], ssems[1])
    scatter_wait(outvs[0], dstvs[0], ssems[0])

    plsc.subcore_barrier()

    @pl.loop(s, NDCH, step=NSUB)
    def _(ch):
        r0 = ch * DROW
        pltpu.sync_copy(acc.at[pl.ds(r0, DROW)],
                        out_hbm.at[pl.ds(c * N + r0, DROW)])


def kernel(x_real, x_imag, edge_index, l_real_w, l_imag_w, weight, bias):
    dst = edge_index[0]
    src = edge_index[1]

    wsel = jnp.stack([
        jnp.concatenate([weight[0][:, :H], weight[1][:, :H], weight[2][:, :H]], axis=1),
        jnp.concatenate([weight[0][:, H:], weight[1][:, H:], weight[2][:, H:]], axis=1),
    ])

    wsel = wsel[:, :, _PERM192]

    ycat = _project(x_real, x_imag, wsel).reshape(NCORE * N, 4 * H, 2)
    ycat = lax.bitcast_convert_type(ycat, jnp.int32)

    out = _sc_spmm(ycat, l_real_w, l_imag_w, src, dst).reshape(NCORE, N, C)

    real = jnp.concatenate([out[0, :, :H], out[1, :, :H]], axis=1) + bias
    imag = jnp.concatenate([out[0, :, H:], out[1, :, H:]], axis=1) + bias
    return (real, imag)
